# baseline (speedup 1.0000x reference)
import functools

import jax
import jax.numpy as jnp
from jax import lax
from jax.experimental import pallas as pl
from jax.experimental.pallas import tpu as pltpu

_BN_EPS = 1e-5


def _conv_acc(slab_ref, w_ref, *, ho, wo, kh, kw, c2):
    m = ho * wo
    acc = None
    for i in range(kh):
        for j in range(kw):
            lhs = slab_ref[0, i:i + ho, j:j + wo, :].reshape(m, c2)
            part = jnp.dot(lhs, w_ref[i * kw + j],
                           preferred_element_type=jnp.float32)
            acc = part if acc is None else acc + part
    return acc


def _stats_kernel(slab_ref, w_ref, stats_ref, *, ho, wo, kh, kw, c2):
    acc = _conv_acc(slab_ref, w_ref, ho=ho, wo=wo, kh=kh, kw=kw, c2=c2)
    stats_ref[0] = jnp.concatenate(
        [jnp.sum(acc, axis=0, keepdims=True),
         jnp.sum(acc * acc, axis=0, keepdims=True)], axis=0)


def _out_kernel(slab_ref, w_ref, stats_ref, g_ref, b_ref, out_ref, *,
                ho, wo, kh, kw, c2, co, m_total):
    acc = _conv_acc(slab_ref, w_ref, ho=ho, wo=wo, kh=kh, kw=kw, c2=c2)
    s = jnp.sum(stats_ref[...], axis=0)
    s = s[:, :co] + s[:, co:]
    mean = s[0:1] / m_total
    var = jnp.maximum(s[1:2] / m_total - mean * mean, 0.0)
    scale = g_ref[...] * lax.rsqrt(var + _BN_EPS)
    shift = b_ref[...] - mean * scale
    ya = jnp.maximum(acc[:, :co] * scale + shift, 0.0)
    yb = jnp.maximum(acc[:, co:] * scale + shift, 0.0)
    out_ref[0] = jnp.transpose(ya, (1, 0))
    out_ref[1] = jnp.transpose(yb, (1, 0))


@jax.jit
def _conv_bn_relu(x, weight, gamma, beta):
    n, c, h, w = x.shape
    co, _, kh, kw = weight.shape
    ho, wo = h, w
    m = ho * wo
    m_total = n * m
    npair = n // 2

    xp = x.reshape(npair, 2, c, h, w).transpose(0, 3, 4, 1, 2)
    xp = xp.reshape(npair, h, w, 2 * c)
    slab = jnp.pad(xp, ((0, 0), (1, 1), (1, 1), (0, 0))).astype(jnp.bfloat16)

    w_t = jnp.transpose(weight, (2, 3, 1, 0)).reshape(kh * kw, c, co)
    zero = jnp.zeros_like(w_t)
    w_bd = jnp.concatenate(
        [jnp.concatenate([w_t, zero], axis=2),
         jnp.concatenate([zero, w_t], axis=2)], axis=1).astype(jnp.bfloat16)
    g2 = gamma.reshape(1, co)
    b2 = beta.reshape(1, co)

    slab_spec = pl.BlockSpec((1, h + kh - 1, w + kw - 1, 2 * c),
                             lambda nb: (nb, 0, 0, 0))
    w_spec = pl.BlockSpec((kh * kw, 2 * c, 2 * co), lambda nb: (0, 0, 0))
    statics = dict(ho=ho, wo=wo, kh=kh, kw=kw, c2=2 * c)
    cparams = pltpu.CompilerParams(dimension_semantics=("parallel",))

    stats = pl.pallas_call(
        functools.partial(_stats_kernel, **statics),
        out_shape=jax.ShapeDtypeStruct((npair, 2, 2 * co), jnp.float32),
        grid=(npair,),
        in_specs=[slab_spec, w_spec],
        out_specs=pl.BlockSpec((1, 2, 2 * co), lambda nb: (nb, 0, 0)),
        compiler_params=cparams,
    )(slab, w_bd)

    out_cm = pl.pallas_call(
        functools.partial(_out_kernel, **statics, co=co, m_total=m_total),
        out_shape=jax.ShapeDtypeStruct((n, co, m), jnp.float32),
        grid=(npair,),
        in_specs=[slab_spec, w_spec,
                  pl.BlockSpec((npair, 2, 2 * co), lambda nb: (0, 0, 0)),
                  pl.BlockSpec((1, co), lambda nb: (0, 0)),
                  pl.BlockSpec((1, co), lambda nb: (0, 0))],
        out_specs=pl.BlockSpec((2, co, m), lambda nb: (nb, 0, 0)),
        compiler_params=cparams,
    )(slab, w_bd, stats, g2, b2)

    return out_cm.reshape(n, co, ho, wo)


def kernel(x, weight, bias, gamma, beta):
    del bias
    return _conv_bn_relu(x, weight, gamma, beta)


# speedup vs baseline: 1.5966x; 1.5966x over previous
import functools

import jax
import jax.numpy as jnp
from jax import lax
from jax.experimental import pallas as pl
from jax.experimental.pallas import tpu as pltpu

_BN_EPS = 1e-5


def _conv_acc(slab_ref, w_ref, *, ho, wo, kh, kw, c2):
    packed = jnp.concatenate([slab_ref[0], slab_ref[1]], axis=-1)
    m = ho * wo
    acc = None
    for i in range(kh):
        for j in range(kw):
            lhs = packed[i:i + ho, j:j + wo, :].reshape(m, c2)
            part = jnp.dot(lhs, w_ref[i * kw + j],
                           preferred_element_type=jnp.float32)
            acc = part if acc is None else acc + part
    return acc


def _stats_kernel(slab_ref, w_ref, stats_ref, *, ho, wo, kh, kw, c2):
    acc = _conv_acc(slab_ref, w_ref, ho=ho, wo=wo, kh=kh, kw=kw, c2=c2)
    stats_ref[0] = jnp.concatenate(
        [jnp.sum(acc, axis=0, keepdims=True),
         jnp.sum(acc * acc, axis=0, keepdims=True)], axis=0)


def _out_kernel(slab_ref, w_ref, stats_ref, g_ref, b_ref, out_ref, *,
                ho, wo, kh, kw, c2, co, m_total):
    acc = _conv_acc(slab_ref, w_ref, ho=ho, wo=wo, kh=kh, kw=kw, c2=c2)
    s = jnp.sum(stats_ref[...], axis=0)
    s = s[:, :co] + s[:, co:]
    mean = s[0:1] / m_total
    var = jnp.maximum(s[1:2] / m_total - mean * mean, 0.0)
    scale = g_ref[...] * lax.rsqrt(var + _BN_EPS)
    shift = b_ref[...] - mean * scale
    ya = jnp.maximum(acc[:, :co] * scale + shift, 0.0)
    yb = jnp.maximum(acc[:, co:] * scale + shift, 0.0)
    out_ref[0] = jnp.transpose(ya, (1, 0))
    out_ref[1] = jnp.transpose(yb, (1, 0))


@jax.jit
def _conv_bn_relu(x, weight, gamma, beta):
    n, c, h, w = x.shape
    co, _, kh, kw = weight.shape
    ho, wo = h, w
    m = ho * wo
    m_total = n * m
    npair = n // 2

    x_nhwc = jnp.transpose(x, (0, 2, 3, 1))
    slab = jnp.pad(x_nhwc, ((0, 0), (1, 1), (1, 1), (0, 0))).astype(jnp.bfloat16)

    w_t = jnp.transpose(weight, (2, 3, 1, 0)).reshape(kh * kw, c, co)
    zero = jnp.zeros_like(w_t)
    w_bd = jnp.concatenate(
        [jnp.concatenate([w_t, zero], axis=2),
         jnp.concatenate([zero, w_t], axis=2)], axis=1).astype(jnp.bfloat16)
    g2 = gamma.reshape(1, co)
    b2 = beta.reshape(1, co)

    slab_spec = pl.BlockSpec((2, h + kh - 1, w + kw - 1, c),
                             lambda nb: (nb, 0, 0, 0))
    w_spec = pl.BlockSpec((kh * kw, 2 * c, 2 * co), lambda nb: (0, 0, 0))
    statics = dict(ho=ho, wo=wo, kh=kh, kw=kw, c2=2 * c)
    cparams = pltpu.CompilerParams(dimension_semantics=("parallel",))

    stats = pl.pallas_call(
        functools.partial(_stats_kernel, **statics),
        out_shape=jax.ShapeDtypeStruct((npair, 2, 2 * co), jnp.float32),
        grid=(npair,),
        in_specs=[slab_spec, w_spec],
        out_specs=pl.BlockSpec((1, 2, 2 * co), lambda nb: (nb, 0, 0)),
        compiler_params=cparams,
    )(slab, w_bd)

    out_cm = pl.pallas_call(
        functools.partial(_out_kernel, **statics, co=co, m_total=m_total),
        out_shape=jax.ShapeDtypeStruct((n, co, m), jnp.float32),
        grid=(npair,),
        in_specs=[slab_spec, w_spec,
                  pl.BlockSpec((npair, 2, 2 * co), lambda nb: (0, 0, 0)),
                  pl.BlockSpec((1, co), lambda nb: (0, 0)),
                  pl.BlockSpec((1, co), lambda nb: (0, 0))],
        out_specs=pl.BlockSpec((2, co, m), lambda nb: (nb, 0, 0)),
        compiler_params=cparams,
    )(slab, w_bd, stats, g2, b2)

    return out_cm.reshape(n, co, ho, wo)


def kernel(x, weight, bias, gamma, beta):
    del bias
    return _conv_bn_relu(x, weight, gamma, beta)
